# Initial kernel scaffold; baseline (speedup 1.0000x reference)
import jax
import jax.numpy as jnp
from jax import lax
from jax.experimental import pallas as pl
from jax.experimental.pallas import tpu as pltpu
from jax.experimental.pallas import tpu_sc as plsc

N = 10000
E = 320000
D = 128

NC = 2
NS = 16
NW = NC * NS
C = 128
CH = 79
T = CH * C
EP = NW * T
NP = 10112
RPT = NP // NS

_mesh = plsc.VectorSubcoreMesh(
    core_axis_name="c", subcore_axis_name="s", num_cores=NC, num_subcores=NS
)

_ROW_PLAN = ((0, 128), (128, 128), (256, 128), (384, 128), (512, 120))


def _deg_body(src3, dst3, degp, src_v, dst_v, ones_v, wb_v, outdeg_s, indeg_s):
    c = lax.axis_index("c")
    s = lax.axis_index("s")
    w = s * NC + c
    pltpu.sync_copy(src3.at[w], src_v)
    pltpu.sync_copy(dst3.at[w], dst_v)

    @pl.loop(0, C // 16)
    def _(i):
        ones_v[pl.ds(i * 16, 16)] = jnp.full((16,), 1.0, jnp.float32)

    @pl.loop(0, 640 // 16)
    def _(i):
        wb_v[pl.ds(i * 16, 16)] = jnp.zeros((16,), jnp.float32)

    pltpu.sync_copy(wb_v.at[pl.ds(0, RPT)], outdeg_s.at[pl.ds(s * RPT, RPT)])
    pltpu.sync_copy(wb_v.at[pl.ds(0, RPT)], indeg_s.at[pl.ds(s * RPT, RPT)])
    plsc.subcore_barrier()

    @pl.loop(0, CH)
    def _(j):
        pltpu.sync_copy(ones_v, outdeg_s.at[src_v.at[j]], add=True)
        pltpu.sync_copy(ones_v, indeg_s.at[dst_v.at[j]], add=True)

    plsc.subcore_barrier()
    pltpu.sync_copy(outdeg_s.at[pl.ds(s * RPT, RPT)], wb_v.at[pl.ds(0, RPT)])
    pltpu.sync_copy(wb_v.at[pl.ds(0, RPT)], degp.at[c, 0, pl.ds(s * RPT, RPT)])
    pltpu.sync_copy(indeg_s.at[pl.ds(s * RPT, RPT)], wb_v.at[pl.ds(0, RPT)])
    pltpu.sync_copy(wb_v.at[pl.ds(0, RPT)], degp.at[c, 1, pl.ds(s * RPT, RPT)])


_deg = pl.kernel(
    _deg_body,
    out_type=jax.ShapeDtypeStruct((NC, 2, NP), jnp.float32),
    mesh=_mesh,
    scratch_types=[
        pltpu.VMEM((CH, C), jnp.int32),
        pltpu.VMEM((CH, C), jnp.int32),
        pltpu.VMEM((C,), jnp.float32),
        pltpu.VMEM((640,), jnp.float32),
        pltpu.VMEM_SHARED((NP,), jnp.float32),
        pltpu.VMEM_SHARED((NP,), jnp.float32),
    ],
)


def _fc_body(x_ref, w_ref, b_ref, degp_ref, h_ref):
    od = degp_ref[0, 0, :] + degp_ref[1, 0, :]
    os = lax.rsqrt(jnp.maximum(od, 1.0))[:, None]
    h = jnp.dot(x_ref[...], w_ref[...], preferred_element_type=jnp.float32)
    h_ref[...] = (h + b_ref[...][None, :]) * os


def _fc(x_pad, w, b, degp):
    return pl.pallas_call(
        _fc_body,
        out_shape=jax.ShapeDtypeStruct((NP, D), jnp.float32),
    )(x_pad, w, b, degp)


def _agg_body(h_hbm, src3, dst3, pp, src_v, dst_v, ebuf, acc_s):
    c = lax.axis_index("c")
    s = lax.axis_index("s")
    w = s * NC + c
    pltpu.sync_copy(src3.at[w], src_v)
    pltpu.sync_copy(dst3.at[w], dst_v)

    @pl.loop(0, C)
    def _(r):
        for cc in range(D // 16):
            ebuf[r, pl.ds(cc * 16, 16)] = jnp.zeros((16,), jnp.float32)

    for off, sz in _ROW_PLAN:
        pltpu.sync_copy(ebuf.at[pl.ds(0, sz)], acc_s.at[pl.ds(s * RPT + off, sz)])
    plsc.subcore_barrier()

    @pl.loop(0, CH)
    def _(j):
        pltpu.sync_copy(h_hbm.at[src_v.at[j]], ebuf)
        pltpu.sync_copy(ebuf, acc_s.at[dst_v.at[j]], add=True)

    plsc.subcore_barrier()
    for off, sz in _ROW_PLAN:
        pltpu.sync_copy(acc_s.at[pl.ds(s * RPT + off, sz)], ebuf.at[pl.ds(0, sz)])
        pltpu.sync_copy(ebuf.at[pl.ds(0, sz)], pp.at[c, pl.ds(s * RPT + off, sz)])


_agg = pl.kernel(
    _agg_body,
    out_type=jax.ShapeDtypeStruct((NC, NP, D), jnp.float32),
    mesh=_mesh,
    scratch_types=[
        pltpu.VMEM((CH, C), jnp.int32),
        pltpu.VMEM((CH, C), jnp.int32),
        pltpu.VMEM((C, D), jnp.float32),
        pltpu.VMEM_SHARED((NP, D), jnp.float32),
    ],
)


def _comb_body(pp_ref, degp_ref, x_ref, out_ref):
    idg = degp_ref[0, 1, :] + degp_ref[1, 1, :]
    isc = lax.rsqrt(jnp.maximum(idg, 1.0))[:, None]
    out_ref[...] = (pp_ref[0] + pp_ref[1]) * isc + x_ref[...]


def _comb(pp, degp, x_pad):
    return pl.pallas_call(
        _comb_body,
        out_shape=jax.ShapeDtypeStruct((NP, D), jnp.float32),
    )(pp, degp, x_pad)


@jax.jit
def kernel(x, edge_index, W, b):
    pad = jnp.full((EP - E,), N, dtype=jnp.int32)
    src3 = jnp.concatenate([edge_index[0], pad]).reshape(NW, CH, C)
    dst3 = jnp.concatenate([edge_index[1], pad]).reshape(NW, CH, C)
    x_pad = jnp.pad(x, ((0, NP - N), (0, 0)))
    degp = _deg(src3, dst3)
    h = _fc(x_pad, W, b, degp)
    pp = _agg(h, src3, dst3)
    rst = _comb(pp, degp, x_pad)
    return rst[:N]


# speedup vs baseline: 6.4792x; 6.4792x over previous
import jax
import jax.numpy as jnp
from jax import lax
from jax.experimental import pallas as pl
from jax.experimental.pallas import tpu as pltpu
from jax.experimental.pallas import tpu_sc as plsc

N = 10000
E = 320000
D = 128

NC = 2
NS = 16
NW = NC * NS
C = 128
CH = 79
T = CH * C
EP = NW * T
NP = 10112
RPT = NP // NS

_mesh = plsc.VectorSubcoreMesh(
    core_axis_name="c", subcore_axis_name="s", num_cores=NC, num_subcores=NS
)

_ROW_PLAN = ((0, 128), (128, 128), (256, 128), (384, 128), (512, 120))


def _deg_body(src3, dst3, degp, src_v, dst_v, ones_v, wb_v, outdeg_s, indeg_s):
    c = lax.axis_index("c")
    s = lax.axis_index("s")
    w = s * NC + c
    pltpu.sync_copy(src3.at[w], src_v)
    pltpu.sync_copy(dst3.at[w], dst_v)

    @pl.loop(0, C // 16)
    def _(i):
        ones_v[pl.ds(i * 16, 16)] = jnp.full((16,), 1.0, jnp.float32)

    @pl.loop(0, 640 // 16)
    def _(i):
        wb_v[pl.ds(i * 16, 16)] = jnp.zeros((16,), jnp.float32)

    pltpu.sync_copy(wb_v.at[pl.ds(0, RPT)], outdeg_s.at[pl.ds(s * RPT, RPT)])
    pltpu.sync_copy(wb_v.at[pl.ds(0, RPT)], indeg_s.at[pl.ds(s * RPT, RPT)])
    plsc.subcore_barrier()

    @pl.loop(0, CH)
    def _(j):
        pltpu.sync_copy(ones_v, outdeg_s.at[src_v.at[j]], add=True)
        pltpu.sync_copy(ones_v, indeg_s.at[dst_v.at[j]], add=True)

    plsc.subcore_barrier()
    pltpu.sync_copy(outdeg_s.at[pl.ds(s * RPT, RPT)], wb_v.at[pl.ds(0, RPT)])
    pltpu.sync_copy(
        wb_v.at[pl.ds(0, RPT)], degp.at[pl.ds(c * 2 * NP + s * RPT, RPT)]
    )
    pltpu.sync_copy(indeg_s.at[pl.ds(s * RPT, RPT)], wb_v.at[pl.ds(0, RPT)])
    pltpu.sync_copy(
        wb_v.at[pl.ds(0, RPT)], degp.at[pl.ds((c * 2 + 1) * NP + s * RPT, RPT)]
    )


_deg = pl.kernel(
    _deg_body,
    out_type=jax.ShapeDtypeStruct((NC * 2 * NP,), jnp.float32),
    mesh=_mesh,
    scratch_types=[
        pltpu.VMEM((CH, C), jnp.int32),
        pltpu.VMEM((CH, C), jnp.int32),
        pltpu.VMEM((C,), jnp.float32),
        pltpu.VMEM((640,), jnp.float32),
        pltpu.VMEM_SHARED((NP,), jnp.float32),
        pltpu.VMEM_SHARED((NP,), jnp.float32),
    ],
)


def _fc_body(x_ref, w_ref, b_ref, degp_ref, h_ref):
    od = degp_ref[0, 0, :] + degp_ref[1, 0, :]
    os = lax.rsqrt(jnp.maximum(od, 1.0))[:, None]
    h = jnp.dot(x_ref[...], w_ref[...], preferred_element_type=jnp.float32)
    h_ref[...] = (h + b_ref[...][None, :]) * os


def _fc(x_pad, w, b, degp):
    return pl.pallas_call(
        _fc_body,
        out_shape=jax.ShapeDtypeStruct((NP, D), jnp.float32),
    )(x_pad, w, b, degp)


def _agg_body(h_hbm, src3, dst3, pp, src_v, dst_v, ebuf, acc_s):
    c = lax.axis_index("c")
    s = lax.axis_index("s")
    w = s * NC + c
    pltpu.sync_copy(src3.at[w], src_v)
    pltpu.sync_copy(dst3.at[w], dst_v)

    @pl.loop(0, C)
    def _(r):
        for cc in range(D // 16):
            ebuf[r, pl.ds(cc * 16, 16)] = jnp.zeros((16,), jnp.float32)

    for off, sz in _ROW_PLAN:
        pltpu.sync_copy(ebuf.at[pl.ds(0, sz)], acc_s.at[pl.ds(s * RPT + off, sz)])
    plsc.subcore_barrier()

    @pl.loop(0, CH)
    def _(j):
        pltpu.sync_copy(h_hbm.at[src_v.at[j]], ebuf)
        pltpu.sync_copy(ebuf, acc_s.at[dst_v.at[j]], add=True)

    plsc.subcore_barrier()
    for off, sz in _ROW_PLAN:
        pltpu.sync_copy(acc_s.at[pl.ds(s * RPT + off, sz)], ebuf.at[pl.ds(0, sz)])
        pltpu.sync_copy(ebuf.at[pl.ds(0, sz)], pp.at[c, pl.ds(s * RPT + off, sz)])


_agg = pl.kernel(
    _agg_body,
    out_type=jax.ShapeDtypeStruct((NC, NP, D), jnp.float32),
    mesh=_mesh,
    scratch_types=[
        pltpu.VMEM((CH, C), jnp.int32),
        pltpu.VMEM((CH, C), jnp.int32),
        pltpu.VMEM((C, D), jnp.float32),
        pltpu.VMEM_SHARED((NP, D), jnp.float32),
    ],
)


def _comb_body(pp_ref, degp_ref, x_ref, out_ref):
    idg = degp_ref[0, 1, :] + degp_ref[1, 1, :]
    isc = lax.rsqrt(jnp.maximum(idg, 1.0))[:, None]
    out_ref[...] = (pp_ref[0] + pp_ref[1]) * isc + x_ref[...]


def _comb(pp, degp, x_pad):
    return pl.pallas_call(
        _comb_body,
        out_shape=jax.ShapeDtypeStruct((NP, D), jnp.float32),
    )(pp, degp, x_pad)


@jax.jit
def kernel(x, edge_index, W, b):
    pad = jnp.full((EP - E,), N, dtype=jnp.int32)
    src3 = jnp.concatenate([edge_index[0], pad]).reshape(NW, CH, C)
    dst3 = jnp.concatenate([edge_index[1], pad]).reshape(NW, CH, C)
    x_pad = jnp.pad(x, ((0, NP - N), (0, 0)))
    degp = _deg(src3, dst3).reshape(NC, 2, NP)
    h = _fc(x_pad, W, b, degp)
    pp = _agg(h, src3, dst3)
    rst = _comb(pp, degp, x_pad)
    return rst[:N]
